# baseline (speedup 1.0000x reference)
import functools
import math

import jax
import jax.numpy as jnp
from jax import lax
from jax.experimental import pallas as pl
from jax.experimental.pallas import tpu as pltpu
from jax.experimental.pallas import tpu_sc as plsc

D = 64
LANES = 16
NCORE = 2
NSUB = 16
NW = NCORE * NSUB

B_TOTAL = 4096 * 200
B_PER_W = B_TOTAL // NW
CHUNK = 320
NCHUNK = B_PER_W // CHUNK
NOUTER = NCHUNK // 2

SCALE = math.sqrt(D)


def _sc_body(table_hbm, idx_hbm, out_hbm,
             idx_v, r0, r1, s0, s1, g0, g1, o0, o1):
    rows = (r0, r1)
    stage = (s0, s1)
    gsem = (g0, g1)
    osem = (o0, o1)

    c = lax.axis_index("c")
    s = lax.axis_index("s")
    wid = s * NCORE + c
    base = wid * B_PER_W

    pltpu.sync_copy(idx_hbm.at[pl.ds(base, B_PER_W)], idx_v)

    def start_gather(ch, b):
        pltpu.async_copy(
            table_hbm.at[idx_v.at[pl.ds(ch * CHUNK, CHUNK)]], rows[b], gsem[b]
        )

    def wait_gather(b):
        pltpu.make_async_copy(
            out_hbm.at[pl.ds(0, CHUNK)], rows[b], gsem[b]
        ).wait()

    def start_store(ch, b):
        pltpu.async_copy(
            stage[b], out_hbm.at[pl.ds(base + ch * CHUNK, CHUNK)], osem[b]
        )

    def wait_store(b):
        pltpu.make_async_copy(
            stage[b], out_hbm.at[pl.ds(0, CHUNK)], osem[b]
        ).wait()

    def scale_chunk(b):
        src = rows[b]
        dst = stage[b]

        @pl.loop(0, CHUNK, unroll=8)
        def _row(r):
            for j in range(D // LANES):
                sl = pl.ds(j * LANES, LANES)
                dst[r, sl] = src[r, sl] * SCALE

    start_gather(0, 0)
    start_gather(1, 1)

    for b in range(2):
        wait_gather(b)
        scale_chunk(b)
        start_store(b, b)
        start_gather(b + 2, b)

    @pl.loop(1, NOUTER - 1)
    def _outer(i):
        for b in range(2):
            ch = i * 2 + b
            wait_gather(b)
            wait_store(b)
            scale_chunk(b)
            start_store(ch, b)
            start_gather(ch + 2, b)

    for b in range(2):
        ch = NCHUNK - 2 + b
        wait_gather(b)
        wait_store(b)
        scale_chunk(b)
        start_store(ch, b)

    wait_store(0)
    wait_store(1)


@jax.jit
def _embed(idx_flat, table):
    mesh = plsc.VectorSubcoreMesh(
        core_axis_name="c", subcore_axis_name="s",
        num_cores=NCORE, num_subcores=NSUB,
    )
    run = functools.partial(
        pl.kernel,
        out_type=jax.ShapeDtypeStruct((B_TOTAL, D), jnp.float32),
        mesh=mesh,
        scratch_types=[
            pltpu.VMEM((B_PER_W,), jnp.int32),
            pltpu.VMEM((CHUNK, D), jnp.float32),
            pltpu.VMEM((CHUNK, D), jnp.float32),
            pltpu.VMEM((CHUNK, D), jnp.float32),
            pltpu.VMEM((CHUNK, D), jnp.float32),
            pltpu.SemaphoreType.DMA,
            pltpu.SemaphoreType.DMA,
            pltpu.SemaphoreType.DMA,
            pltpu.SemaphoreType.DMA,
        ],
        compiler_params=pltpu.CompilerParams(use_tc_tiling_on_sc=False),
    )(_sc_body)
    return run(table, idx_flat)


def kernel(input_, table):
    idx_flat = input_.reshape(-1).astype(jnp.int32)
    out = _embed(idx_flat, table)
    return out.reshape(*input_.shape, D)
